# baseline (speedup 1.0000x reference)
import functools

import jax
import jax.numpy as jnp
from jax import lax
from jax.experimental import pallas as pl
from jax.experimental.pallas import tpu as pltpu
from jax.experimental.pallas import tpu_sc as plsc

_NUM_PROJ = 32
_EPS = 1e-06
_B, _T, _D = 4, 4096, 1024
_TB = 2048
_NW = 32
_NV = _T // 16
_NPAIR = _T // 32
_NCHUNK = 4
_BC = _B // _NCHUNK
_CROWS = _BC * _NUM_PROJ
_PPW = _CROWS // _NW
_LSTEP = 2 * _PPW


def _dot_lt(a, b):
    return lax.dot_general(
        a, b, (((0,), (1,)), ((), ())),
        preferred_element_type=jnp.float32)


def _proj_body(x_ref, y_ref, p_ref, px_ref, py_ref):
    p = p_ref[...]
    pss = jnp.sum(p * p, axis=0, keepdims=True)
    pn = p / jnp.maximum(jnp.sqrt(pss), _EPS)
    pnh = pn.astype(jnp.bfloat16)
    for src, dst in ((x_ref, px_ref), (y_ref, py_ref)):
        v = src[0]
        ss = jnp.sum(v * v, axis=1)
        s = 1.0 / jnp.maximum(jnp.sqrt(ss), _EPS)
        out = _dot_lt(pnh, v.astype(jnp.bfloat16))
        dst[...] = out * s[None, :]


def _project(x, y, projections, chunk):
    grid = (_BC, _T // _TB)
    boff = chunk * _BC
    return pl.pallas_call(
        _proj_body,
        grid=grid,
        in_specs=[
            pl.BlockSpec((1, _TB, _D), lambda b, t: (b + boff, t, 0)),
            pl.BlockSpec((1, _TB, _D), lambda b, t: (b + boff, t, 0)),
            pl.BlockSpec((_D, _NUM_PROJ), lambda b, t: (0, 0)),
        ],
        out_specs=[
            pl.BlockSpec((_NUM_PROJ, _TB), lambda b, t: (b, t)),
            pl.BlockSpec((_NUM_PROJ, _TB), lambda b, t: (b, t)),
        ],
        out_shape=[
            jax.ShapeDtypeStruct((_CROWS, _T), jnp.float32),
            jax.ShapeDtypeStruct((_CROWS, _T), jnp.float32),
        ],
    )(x, y, projections)


def _sc_cost_body(px_hbm, py_hbm, out_hbm, buf, costref):
    c = lax.axis_index("c")
    s = lax.axis_index("s")
    w = s * 2 + c
    base = w * _PPW
    pltpu.sync_copy(px_hbm.at[pl.ds(base, _PPW)], buf.at[pl.ds(0, _PPW)])
    pltpu.sync_copy(py_hbm.at[pl.ds(base, _PPW)], buf.at[pl.ds(_PPW, _PPW)])

    @plsc.parallel_loop(0, _NPAIR, 1, unroll=1)
    def _lvl16(pp):
        ai = pp * 32
        bi = ai + 16
        for r in range(_LSTEP):
            a = jnp.sort(buf[r, pl.ds(ai, 16)])
            b = jnp.sort(buf[r, pl.ds(bi, 16)])
            bv = jnp.flip(b, axis=0)
            buf[r, pl.ds(ai, 16)] = jnp.sort(jnp.minimum(a, bv))
            buf[r, pl.ds(bi, 16)] = jnp.sort(jnp.maximum(a, bv))

    for n in (32, 64, 128, 256, 512, 1024, 2048):
        nb = n // 16

        @plsc.parallel_loop(0, _NPAIR, 1, unroll=1)
        def _cross(p, nb=nb, n=n):
            blk = p // nb
            r16 = p % nb
            ai = blk * (2 * n) + r16 * 16
            bi = blk * (2 * n) + 2 * n - r16 * 16 - 16
            for r in range(_LSTEP):
                av = buf[r, pl.ds(ai, 16)]
                bv = jnp.flip(buf[r, pl.ds(bi, 16)], axis=0)
                buf[r, pl.ds(ai, 16)] = jnp.minimum(av, bv)
                buf[r, pl.ds(bi, 16)] = jnp.flip(jnp.maximum(av, bv), axis=0)

        d = n // 2
        while d >= 32:
            q = d // 16

            @plsc.parallel_loop(0, _NPAIR, 1, unroll=1)
            def _stage(p, q=q, d=d):
                blk = p // q
                r16 = p % q
                i0 = blk * (2 * d) + r16 * 16
                i1 = i0 + d
                for r in range(_LSTEP):
                    av = buf[r, pl.ds(i0, 16)]
                    bv = buf[r, pl.ds(i1, 16)]
                    buf[r, pl.ds(i0, 16)] = jnp.minimum(av, bv)
                    buf[r, pl.ds(i1, 16)] = jnp.maximum(av, bv)
            d //= 2

        @plsc.parallel_loop(0, _NPAIR, 1, unroll=1)
        def _finish(pp):
            i0 = pp * 32
            i1 = i0 + 16
            for r in range(_LSTEP):
                av = buf[r, pl.ds(i0, 16)]
                bv = buf[r, pl.ds(i1, 16)]
                buf[r, pl.ds(i0, 16)] = jnp.sort(jnp.minimum(av, bv))
                buf[r, pl.ds(i1, 16)] = jnp.sort(jnp.maximum(av, bv))

    lane = lax.iota(jnp.int32, 16)
    costv = jnp.zeros((16,), jnp.float32)
    inv_t = 1.0 / _T
    for i in range(_PPW):
        def acc_body(j, acc, i=i):
            off = j * 16
            dlt = buf[i, pl.ds(off, 16)] - buf[_PPW + i, pl.ds(off, 16)]
            return acc + dlt * dlt
        acc = lax.fori_loop(0, _NV, acc_body, jnp.zeros((16,), jnp.float32))
        s_i = jnp.sum(acc) * inv_t
        costv = costv + jnp.where(lane == i, s_i, 0.0)
    costref[...] = costv
    pltpu.sync_copy(costref, out_hbm.at[w])


def _sc_cost(px, py):
    mesh = plsc.VectorSubcoreMesh(core_axis_name="c", subcore_axis_name="s")
    fn = functools.partial(
        pl.kernel,
        mesh=mesh,
        out_type=jax.ShapeDtypeStruct((_NW, 16), jnp.float32),
        scratch_types=[
            pltpu.VMEM((_LSTEP, _T), jnp.float32),
            pltpu.VMEM((16,), jnp.float32),
        ],
        compiler_params=pltpu.CompilerParams(needs_layout_passes=False),
    )(_sc_cost_body)
    return fn(px, py)


def kernel(x, y, projections):
    costs = []
    for chunk in range(_NCHUNK):
        px, py = _project(x, y, projections, chunk)
        cost = _sc_cost(px, py)
        costs.append(cost[:, :_PPW].reshape(_BC, _NUM_PROJ))
    cost_bl = jnp.concatenate(costs, axis=0)
    return jnp.clip(jnp.mean(cost_bl, axis=-1), _EPS, None) ** 0.5


# speedup vs baseline: 1.3413x; 1.3413x over previous
import functools

import jax
import jax.numpy as jnp
from jax import lax
from jax.experimental import pallas as pl
from jax.experimental.pallas import tpu as pltpu
from jax.experimental.pallas import tpu_sc as plsc

_NUM_PROJ = 32
_EPS = 1e-06
_B, _T, _D = 4, 4096, 1024
_TB = 2048
_NW = 32
_NV = _T // 16
_NPAIR = _T // 32
_NCHUNK = 2
_BC = _B // _NCHUNK
_CROWS = _BC * _NUM_PROJ
_PPW = _CROWS // _NW
_LSTEP = 2 * _PPW


def _dot_lt(a, b):
    return lax.dot_general(
        a, b, (((0,), (1,)), ((), ())),
        preferred_element_type=jnp.float32)


def _proj_body(x_ref, y_ref, p_ref, px_ref, py_ref):
    p = p_ref[...]
    pss = jnp.sum(p * p, axis=0, keepdims=True)
    pn = p / jnp.maximum(jnp.sqrt(pss), _EPS)
    pnh = pn.astype(jnp.bfloat16)
    for src, dst in ((x_ref, px_ref), (y_ref, py_ref)):
        v = src[0]
        ss = jnp.sum(v * v, axis=1)
        s = 1.0 / jnp.maximum(jnp.sqrt(ss), _EPS)
        out = _dot_lt(pnh, v.astype(jnp.bfloat16))
        dst[...] = out * s[None, :]


def _project(x, y, projections, chunk):
    grid = (_BC, _T // _TB)
    boff = chunk * _BC
    return pl.pallas_call(
        _proj_body,
        grid=grid,
        in_specs=[
            pl.BlockSpec((1, _TB, _D), lambda b, t: (b + boff, t, 0)),
            pl.BlockSpec((1, _TB, _D), lambda b, t: (b + boff, t, 0)),
            pl.BlockSpec((_D, _NUM_PROJ), lambda b, t: (0, 0)),
        ],
        out_specs=[
            pl.BlockSpec((_NUM_PROJ, _TB), lambda b, t: (b, t)),
            pl.BlockSpec((_NUM_PROJ, _TB), lambda b, t: (b, t)),
        ],
        out_shape=[
            jax.ShapeDtypeStruct((_CROWS, _T), jnp.float32),
            jax.ShapeDtypeStruct((_CROWS, _T), jnp.float32),
        ],
    )(x, y, projections)


def _sc_cost_body(px_hbm, py_hbm, out_hbm, buf, costref, sem1, sem2):
    c = lax.axis_index("c")
    s = lax.axis_index("s")
    w = s * 2 + c
    base = w * _PPW
    cp1 = pltpu.async_copy(px_hbm.at[pl.ds(base, _PPW)],
                           buf.at[pl.ds(0, _PPW)], sem1)
    cp2 = pltpu.async_copy(py_hbm.at[pl.ds(base, _PPW)],
                           buf.at[pl.ds(_PPW, _PPW)], sem2)
    cp1.wait()
    cp2.wait()

    @plsc.parallel_loop(0, _NPAIR, 1, unroll=1)
    def _lvl16(pp):
        ai = pp * 32
        bi = ai + 16
        for r in range(_LSTEP):
            a = jnp.sort(buf[r, pl.ds(ai, 16)])
            b = jnp.sort(buf[r, pl.ds(bi, 16)])
            bv = jnp.flip(b, axis=0)
            buf[r, pl.ds(ai, 16)] = jnp.sort(jnp.minimum(a, bv))
            buf[r, pl.ds(bi, 16)] = jnp.sort(jnp.maximum(a, bv))

    accs = None
    for n in (32, 64, 128, 256, 512, 1024, 2048):
        nb = n // 16

        @plsc.parallel_loop(0, _NPAIR, 1, unroll=1)
        def _cross(p, nb=nb, n=n):
            blk = p // nb
            r16 = p % nb
            ai = blk * (2 * n) + r16 * 16
            bi = blk * (2 * n) + 2 * n - r16 * 16 - 16
            for r in range(_LSTEP):
                av = buf[r, pl.ds(ai, 16)]
                bv = jnp.flip(buf[r, pl.ds(bi, 16)], axis=0)
                buf[r, pl.ds(ai, 16)] = jnp.minimum(av, bv)
                buf[r, pl.ds(bi, 16)] = jnp.flip(jnp.maximum(av, bv), axis=0)

        d = n // 2
        while d >= 32:
            q = d // 16

            @plsc.parallel_loop(0, _NPAIR, 1, unroll=1)
            def _stage(p, q=q, d=d):
                blk = p // q
                r16 = p % q
                i0 = blk * (2 * d) + r16 * 16
                i1 = i0 + d
                for r in range(_LSTEP):
                    av = buf[r, pl.ds(i0, 16)]
                    bv = buf[r, pl.ds(i1, 16)]
                    buf[r, pl.ds(i0, 16)] = jnp.minimum(av, bv)
                    buf[r, pl.ds(i1, 16)] = jnp.maximum(av, bv)
            d //= 2

        if n < 2048:
            @plsc.parallel_loop(0, _NPAIR, 1, unroll=1)
            def _finish(pp):
                i0 = pp * 32
                i1 = i0 + 16
                for r in range(_LSTEP):
                    av = buf[r, pl.ds(i0, 16)]
                    bv = buf[r, pl.ds(i1, 16)]
                    buf[r, pl.ds(i0, 16)] = jnp.sort(jnp.minimum(av, bv))
                    buf[r, pl.ds(i1, 16)] = jnp.sort(jnp.maximum(av, bv))
        else:
            init = tuple(jnp.zeros((16,), jnp.float32) for _ in range(2 * _PPW))

            @plsc.parallel_loop(0, _NPAIR, 1, unroll=1, carry=init)
            def _finish_acc(pp, acc):
                i0 = pp * 32
                i1 = i0 + 16
                vals = []
                for r in range(_LSTEP):
                    av = buf[r, pl.ds(i0, 16)]
                    bv = buf[r, pl.ds(i1, 16)]
                    vals.append((jnp.sort(jnp.minimum(av, bv)),
                                 jnp.sort(jnp.maximum(av, bv))))
                new = []
                for i in range(_PPW):
                    d0 = vals[i][0] - vals[_PPW + i][0]
                    d1 = vals[i][1] - vals[_PPW + i][1]
                    new.append(acc[2 * i] + d0 * d0)
                    new.append(acc[2 * i + 1] + d1 * d1)
                return tuple(new)
            accs = _finish_acc

    lane = lax.iota(jnp.int32, 16)
    costv = jnp.zeros((16,), jnp.float32)
    inv_t = 1.0 / _T
    for i in range(_PPW):
        s_i = jnp.sum(accs[2 * i] + accs[2 * i + 1]) * inv_t
        costv = costv + jnp.where(lane == i, s_i, 0.0)
    costref[...] = costv
    pltpu.sync_copy(costref, out_hbm.at[w])


def _sc_cost(px, py):
    mesh = plsc.VectorSubcoreMesh(core_axis_name="c", subcore_axis_name="s")
    fn = functools.partial(
        pl.kernel,
        mesh=mesh,
        out_type=jax.ShapeDtypeStruct((_NW, 16), jnp.float32),
        scratch_types=[
            pltpu.VMEM((_LSTEP, _T), jnp.float32),
            pltpu.VMEM((16,), jnp.float32),
            pltpu.SemaphoreType.DMA,
            pltpu.SemaphoreType.DMA,
        ],
        compiler_params=pltpu.CompilerParams(needs_layout_passes=False),
    )(_sc_cost_body)
    return fn(px, py)


def kernel(x, y, projections):
    costs = []
    for chunk in range(_NCHUNK):
        px, py = _project(x, y, projections, chunk)
        cost = _sc_cost(px, py)
        costs.append(cost[:, :_PPW].reshape(_BC, _NUM_PROJ))
    cost_bl = jnp.concatenate(costs, axis=0)
    return jnp.clip(jnp.mean(cost_bl, axis=-1), _EPS, None) ** 0.5


# speedup vs baseline: 1.4497x; 1.0808x over previous
import functools

import jax
import jax.numpy as jnp
from jax import lax
from jax.experimental import pallas as pl
from jax.experimental.pallas import tpu as pltpu
from jax.experimental.pallas import tpu_sc as plsc

_NUM_PROJ = 32
_EPS = 1e-06
_B, _T, _D = 4, 4096, 1024
_TB = 2048
_NW = 32
_NV = _T // 16
_NPAIR = _T // 32
_NCHUNK = 2
_BC = _B // _NCHUNK
_CROWS = _BC * _NUM_PROJ
_PPW = _CROWS // _NW
_LSTEP = 2 * _PPW


def _dot_lt(a, b):
    return lax.dot_general(
        a, b, (((0,), (1,)), ((), ())),
        preferred_element_type=jnp.float32)


def _proj_body(x_ref, y_ref, p_ref, px_ref, py_ref):
    p = p_ref[...]
    pss = jnp.sum(p * p, axis=0, keepdims=True)
    pn = p / jnp.maximum(jnp.sqrt(pss), _EPS)
    pnh = pn.astype(jnp.bfloat16)
    for src, dst in ((x_ref, px_ref), (y_ref, py_ref)):
        v = src[0]
        ss = jnp.sum(v * v, axis=1)
        s = 1.0 / jnp.maximum(jnp.sqrt(ss), _EPS)
        out = _dot_lt(pnh, v.astype(jnp.bfloat16))
        dst[...] = out * s[None, :]


def _project(x, y, projections, chunk):
    grid = (_BC, _T // _TB)
    boff = chunk * _BC
    return pl.pallas_call(
        _proj_body,
        grid=grid,
        in_specs=[
            pl.BlockSpec((1, _TB, _D), lambda b, t: (b + boff, t, 0)),
            pl.BlockSpec((1, _TB, _D), lambda b, t: (b + boff, t, 0)),
            pl.BlockSpec((_D, _NUM_PROJ), lambda b, t: (0, 0)),
        ],
        out_specs=[
            pl.BlockSpec((_NUM_PROJ, _TB), lambda b, t: (b, t)),
            pl.BlockSpec((_NUM_PROJ, _TB), lambda b, t: (b, t)),
        ],
        out_shape=[
            jax.ShapeDtypeStruct((_CROWS, _T), jnp.float32),
            jax.ShapeDtypeStruct((_CROWS, _T), jnp.float32),
        ],
    )(x, y, projections)


def _sc_cost_body(px_hbm, py_hbm, out_hbm, buf, costref, sem1, sem2):
    c = lax.axis_index("c")
    s = lax.axis_index("s")
    w = s * 2 + c
    base = w * _PPW
    cp1 = pltpu.async_copy(px_hbm.at[pl.ds(base, _PPW)],
                           buf.at[pl.ds(0, _PPW)], sem1)
    cp2 = pltpu.async_copy(py_hbm.at[pl.ds(base, _PPW)],
                           buf.at[pl.ds(_PPW, _PPW)], sem2)
    cp1.wait()
    cp2.wait()

    @plsc.parallel_loop(0, _T // 256, 1, unroll=1)
    def _low(chunk):
        base = chunk * 256
        for r in range(_LSTEP):
            v = [jnp.sort(buf[r, pl.ds(base + 16 * k, 16)]) for k in range(16)]
            for j in range(8):
                a, b = v[2 * j], v[2 * j + 1]
                bv = jnp.flip(b, 0)
                v[2 * j] = jnp.sort(jnp.minimum(a, bv))
                v[2 * j + 1] = jnp.sort(jnp.maximum(a, bv))
            for j in range(4):
                o = 4 * j
                f3 = jnp.flip(v[o + 3], 0)
                f2 = jnp.flip(v[o + 2], 0)
                lo0 = jnp.minimum(v[o], f3)
                hi3 = jnp.flip(jnp.maximum(v[o], f3), 0)
                lo1 = jnp.minimum(v[o + 1], f2)
                hi2 = jnp.flip(jnp.maximum(v[o + 1], f2), 0)
                v[o] = jnp.sort(jnp.minimum(lo0, lo1))
                v[o + 1] = jnp.sort(jnp.maximum(lo0, lo1))
                v[o + 2] = jnp.sort(jnp.minimum(hi2, hi3))
                v[o + 3] = jnp.sort(jnp.maximum(hi2, hi3))
            for j in range(2):
                o = 8 * j
                lo = [None] * 4
                hi = [None] * 4
                for i in range(4):
                    fb = jnp.flip(v[o + 7 - i], 0)
                    lo[i] = jnp.minimum(v[o + i], fb)
                    hi[3 - i] = jnp.flip(jnp.maximum(v[o + i], fb), 0)
                for half in (lo, hi):
                    t0 = jnp.minimum(half[0], half[2])
                    t2 = jnp.maximum(half[0], half[2])
                    t1 = jnp.minimum(half[1], half[3])
                    t3 = jnp.maximum(half[1], half[3])
                    half[0] = jnp.sort(jnp.minimum(t0, t1))
                    half[1] = jnp.sort(jnp.maximum(t0, t1))
                    half[2] = jnp.sort(jnp.minimum(t2, t3))
                    half[3] = jnp.sort(jnp.maximum(t2, t3))
                for i in range(4):
                    v[o + i] = lo[i]
                    v[o + 4 + i] = hi[i]
            lo = [None] * 8
            hi = [None] * 8
            for i in range(8):
                fb = jnp.flip(v[15 - i], 0)
                lo[i] = jnp.minimum(v[i], fb)
                hi[7 - i] = jnp.flip(jnp.maximum(v[i], fb), 0)
            for half in (lo, hi):
                t = [None] * 8
                for i in range(4):
                    t[i] = jnp.minimum(half[i], half[i + 4])
                    t[i + 4] = jnp.maximum(half[i], half[i + 4])
                u = [None] * 8
                for g in (0, 4):
                    u[g] = jnp.minimum(t[g], t[g + 2])
                    u[g + 2] = jnp.maximum(t[g], t[g + 2])
                    u[g + 1] = jnp.minimum(t[g + 1], t[g + 3])
                    u[g + 3] = jnp.maximum(t[g + 1], t[g + 3])
                for g in range(0, 8, 2):
                    half[g] = jnp.sort(jnp.minimum(u[g], u[g + 1]))
                    half[g + 1] = jnp.sort(jnp.maximum(u[g], u[g + 1]))
            v = lo + hi
            for k in range(16):
                buf[r, pl.ds(base + 16 * k, 16)] = v[k]

    accs = None
    for n in (256, 512, 1024, 2048):
        nb = n // 16

        @plsc.parallel_loop(0, _NPAIR, 1, unroll=1)
        def _cross(p, nb=nb, n=n):
            blk = p // nb
            r16 = p % nb
            ai = blk * (2 * n) + r16 * 16
            bi = blk * (2 * n) + 2 * n - r16 * 16 - 16
            for r in range(_LSTEP):
                av = buf[r, pl.ds(ai, 16)]
                bv = jnp.flip(buf[r, pl.ds(bi, 16)], axis=0)
                buf[r, pl.ds(ai, 16)] = jnp.minimum(av, bv)
                buf[r, pl.ds(bi, 16)] = jnp.flip(jnp.maximum(av, bv), axis=0)

        d = n // 2
        while d >= 32:
            q = d // 16

            @plsc.parallel_loop(0, _NPAIR, 1, unroll=1)
            def _stage(p, q=q, d=d):
                blk = p // q
                r16 = p % q
                i0 = blk * (2 * d) + r16 * 16
                i1 = i0 + d
                for r in range(_LSTEP):
                    av = buf[r, pl.ds(i0, 16)]
                    bv = buf[r, pl.ds(i1, 16)]
                    buf[r, pl.ds(i0, 16)] = jnp.minimum(av, bv)
                    buf[r, pl.ds(i1, 16)] = jnp.maximum(av, bv)
            d //= 2

        if n < 2048:
            @plsc.parallel_loop(0, _NPAIR, 1, unroll=1)
            def _finish(pp):
                i0 = pp * 32
                i1 = i0 + 16
                for r in range(_LSTEP):
                    av = buf[r, pl.ds(i0, 16)]
                    bv = buf[r, pl.ds(i1, 16)]
                    buf[r, pl.ds(i0, 16)] = jnp.sort(jnp.minimum(av, bv))
                    buf[r, pl.ds(i1, 16)] = jnp.sort(jnp.maximum(av, bv))
        else:
            init = tuple(jnp.zeros((16,), jnp.float32) for _ in range(2 * _PPW))

            @plsc.parallel_loop(0, _NPAIR, 1, unroll=1, carry=init)
            def _finish_acc(pp, acc):
                i0 = pp * 32
                i1 = i0 + 16
                vals = []
                for r in range(_LSTEP):
                    av = buf[r, pl.ds(i0, 16)]
                    bv = buf[r, pl.ds(i1, 16)]
                    vals.append((jnp.sort(jnp.minimum(av, bv)),
                                 jnp.sort(jnp.maximum(av, bv))))
                new = []
                for i in range(_PPW):
                    d0 = vals[i][0] - vals[_PPW + i][0]
                    d1 = vals[i][1] - vals[_PPW + i][1]
                    new.append(acc[2 * i] + d0 * d0)
                    new.append(acc[2 * i + 1] + d1 * d1)
                return tuple(new)
            accs = _finish_acc

    lane = lax.iota(jnp.int32, 16)
    costv = jnp.zeros((16,), jnp.float32)
    inv_t = 1.0 / _T
    for i in range(_PPW):
        s_i = jnp.sum(accs[2 * i] + accs[2 * i + 1]) * inv_t
        costv = costv + jnp.where(lane == i, s_i, 0.0)
    costref[...] = costv
    pltpu.sync_copy(costref, out_hbm.at[w])


def _sc_cost(px, py):
    mesh = plsc.VectorSubcoreMesh(core_axis_name="c", subcore_axis_name="s")
    fn = functools.partial(
        pl.kernel,
        mesh=mesh,
        out_type=jax.ShapeDtypeStruct((_NW, 16), jnp.float32),
        scratch_types=[
            pltpu.VMEM((_LSTEP, _T), jnp.float32),
            pltpu.VMEM((16,), jnp.float32),
            pltpu.SemaphoreType.DMA,
            pltpu.SemaphoreType.DMA,
        ],
        compiler_params=pltpu.CompilerParams(needs_layout_passes=False),
    )(_sc_cost_body)
    return fn(px, py)


def kernel(x, y, projections):
    costs = []
    for chunk in range(_NCHUNK):
        px, py = _project(x, y, projections, chunk)
        cost = _sc_cost(px, py)
        costs.append(cost[:, :_PPW].reshape(_BC, _NUM_PROJ))
    cost_bl = jnp.concatenate(costs, axis=0)
    return jnp.clip(jnp.mean(cost_bl, axis=-1), _EPS, None) ** 0.5


# speedup vs baseline: 1.5167x; 1.0462x over previous
import functools

import jax
import jax.numpy as jnp
from jax import lax
from jax.experimental import pallas as pl
from jax.experimental.pallas import tpu as pltpu
from jax.experimental.pallas import tpu_sc as plsc

_NUM_PROJ = 32
_EPS = 1e-06
_B, _T, _D = 4, 4096, 1024
_TB = 2048
_NW = 32
_NV = _T // 16
_NPAIR = _T // 32
_NCHUNK = 2
_BC = _B // _NCHUNK
_CROWS = _BC * _NUM_PROJ
_PPW = _CROWS // _NW
_LSTEP = 2 * _PPW


def _dot_lt(a, b):
    return lax.dot_general(
        a, b, (((0,), (1,)), ((), ())),
        preferred_element_type=jnp.float32)


def _proj_body(x_ref, y_ref, p_ref, px_ref, py_ref):
    p = p_ref[...]
    pss = jnp.sum(p * p, axis=0, keepdims=True)
    pn = p / jnp.maximum(jnp.sqrt(pss), _EPS)
    pnh = pn.astype(jnp.bfloat16)
    for src, dst in ((x_ref, px_ref), (y_ref, py_ref)):
        v = src[0]
        ss = jnp.sum(v * v, axis=1)
        s = 1.0 / jnp.maximum(jnp.sqrt(ss), _EPS)
        out = _dot_lt(pnh, v.astype(jnp.bfloat16))
        dst[...] = out * s[None, :]


def _project(x, y, projections, chunk):
    grid = (_BC, _T // _TB)
    boff = chunk * _BC
    return pl.pallas_call(
        _proj_body,
        grid=grid,
        in_specs=[
            pl.BlockSpec((1, _TB, _D), lambda b, t: (b + boff, t, 0)),
            pl.BlockSpec((1, _TB, _D), lambda b, t: (b + boff, t, 0)),
            pl.BlockSpec((_D, _NUM_PROJ), lambda b, t: (0, 0)),
        ],
        out_specs=[
            pl.BlockSpec((_NUM_PROJ, _TB), lambda b, t: (b, t)),
            pl.BlockSpec((_NUM_PROJ, _TB), lambda b, t: (b, t)),
        ],
        out_shape=[
            jax.ShapeDtypeStruct((_CROWS, _T), jnp.float32),
            jax.ShapeDtypeStruct((_CROWS, _T), jnp.float32),
        ],
    )(x, y, projections)


def _sc_cost_body(px_hbm, py_hbm, out_hbm, buf, costref, sem1, sem2):
    c = lax.axis_index("c")
    s = lax.axis_index("s")
    w = s * 2 + c
    base = w * _PPW
    cp1 = pltpu.async_copy(px_hbm.at[pl.ds(base, _PPW)],
                           buf.at[pl.ds(0, _PPW)], sem1)
    cp2 = pltpu.async_copy(py_hbm.at[pl.ds(base, _PPW)],
                           buf.at[pl.ds(_PPW, _PPW)], sem2)
    cp1.wait()
    cp2.wait()

    @plsc.parallel_loop(0, _T // 256, 1, unroll=1)
    def _low(chunk):
        base = chunk * 256
        for r in range(_LSTEP):
            v = [jnp.sort(buf[r, pl.ds(base + 16 * k, 16)]) for k in range(16)]
            for j in range(8):
                a, b = v[2 * j], v[2 * j + 1]
                bv = jnp.flip(b, 0)
                v[2 * j] = jnp.sort(jnp.minimum(a, bv))
                v[2 * j + 1] = jnp.sort(jnp.maximum(a, bv))
            for j in range(4):
                o = 4 * j
                f3 = jnp.flip(v[o + 3], 0)
                f2 = jnp.flip(v[o + 2], 0)
                lo0 = jnp.minimum(v[o], f3)
                hi3 = jnp.flip(jnp.maximum(v[o], f3), 0)
                lo1 = jnp.minimum(v[o + 1], f2)
                hi2 = jnp.flip(jnp.maximum(v[o + 1], f2), 0)
                v[o] = jnp.sort(jnp.minimum(lo0, lo1))
                v[o + 1] = jnp.sort(jnp.maximum(lo0, lo1))
                v[o + 2] = jnp.sort(jnp.minimum(hi2, hi3))
                v[o + 3] = jnp.sort(jnp.maximum(hi2, hi3))
            for j in range(2):
                o = 8 * j
                lo = [None] * 4
                hi = [None] * 4
                for i in range(4):
                    fb = jnp.flip(v[o + 7 - i], 0)
                    lo[i] = jnp.minimum(v[o + i], fb)
                    hi[3 - i] = jnp.flip(jnp.maximum(v[o + i], fb), 0)
                for half in (lo, hi):
                    t0 = jnp.minimum(half[0], half[2])
                    t2 = jnp.maximum(half[0], half[2])
                    t1 = jnp.minimum(half[1], half[3])
                    t3 = jnp.maximum(half[1], half[3])
                    half[0] = jnp.sort(jnp.minimum(t0, t1))
                    half[1] = jnp.sort(jnp.maximum(t0, t1))
                    half[2] = jnp.sort(jnp.minimum(t2, t3))
                    half[3] = jnp.sort(jnp.maximum(t2, t3))
                for i in range(4):
                    v[o + i] = lo[i]
                    v[o + 4 + i] = hi[i]
            lo = [None] * 8
            hi = [None] * 8
            for i in range(8):
                fb = jnp.flip(v[15 - i], 0)
                lo[i] = jnp.minimum(v[i], fb)
                hi[7 - i] = jnp.flip(jnp.maximum(v[i], fb), 0)
            for half in (lo, hi):
                t = [None] * 8
                for i in range(4):
                    t[i] = jnp.minimum(half[i], half[i + 4])
                    t[i + 4] = jnp.maximum(half[i], half[i + 4])
                u = [None] * 8
                for g in (0, 4):
                    u[g] = jnp.minimum(t[g], t[g + 2])
                    u[g + 2] = jnp.maximum(t[g], t[g + 2])
                    u[g + 1] = jnp.minimum(t[g + 1], t[g + 3])
                    u[g + 3] = jnp.maximum(t[g + 1], t[g + 3])
                for g in range(0, 8, 2):
                    half[g] = jnp.sort(jnp.minimum(u[g], u[g + 1]))
                    half[g + 1] = jnp.sort(jnp.maximum(u[g], u[g + 1]))
            v = lo + hi
            for k in range(16):
                buf[r, pl.ds(base + 16 * k, 16)] = v[k]

    def _tail16(v):
        t = [None] * 16
        for i in range(8):
            t[i] = jnp.minimum(v[i], v[i + 8])
            t[i + 8] = jnp.maximum(v[i], v[i + 8])
        out = []
        for h in range(2):
            half = t[8 * h: 8 * h + 8]
            u = [None] * 8
            for i in range(4):
                u[i] = jnp.minimum(half[i], half[i + 4])
                u[i + 4] = jnp.maximum(half[i], half[i + 4])
            ww = [None] * 8
            for g in (0, 4):
                ww[g] = jnp.minimum(u[g], u[g + 2])
                ww[g + 2] = jnp.maximum(u[g], u[g + 2])
                ww[g + 1] = jnp.minimum(u[g + 1], u[g + 3])
                ww[g + 3] = jnp.maximum(u[g + 1], u[g + 3])
            for g in range(0, 8, 2):
                out.append(jnp.sort(jnp.minimum(ww[g], ww[g + 1])))
                out.append(jnp.sort(jnp.maximum(ww[g], ww[g + 1])))
        return out

    accs = None
    for n in (256, 512, 1024, 2048):
        nb = n // 16

        @plsc.parallel_loop(0, _NPAIR, 1, unroll=1)
        def _cross(p, nb=nb, n=n):
            blk = p // nb
            r16 = p % nb
            ai = blk * (2 * n) + r16 * 16
            bi = blk * (2 * n) + 2 * n - r16 * 16 - 16
            for r in range(_LSTEP):
                av = buf[r, pl.ds(ai, 16)]
                bv = jnp.flip(buf[r, pl.ds(bi, 16)], axis=0)
                buf[r, pl.ds(ai, 16)] = jnp.minimum(av, bv)
                buf[r, pl.ds(bi, 16)] = jnp.flip(jnp.maximum(av, bv), axis=0)

        d = n // 2
        while d >= 256:
            q = d // 16

            @plsc.parallel_loop(0, _NPAIR, 1, unroll=1)
            def _stage(p, q=q, d=d):
                blk = p // q
                r16 = p % q
                i0 = blk * (2 * d) + r16 * 16
                i1 = i0 + d
                for r in range(_LSTEP):
                    av = buf[r, pl.ds(i0, 16)]
                    bv = buf[r, pl.ds(i1, 16)]
                    buf[r, pl.ds(i0, 16)] = jnp.minimum(av, bv)
                    buf[r, pl.ds(i1, 16)] = jnp.maximum(av, bv)
            d //= 2

        if n < 2048:
            @plsc.parallel_loop(0, _T // 256, 1, unroll=1)
            def _tailpass(chunk):
                base = chunk * 256
                for r in range(_LSTEP):
                    v = [buf[r, pl.ds(base + 16 * k, 16)] for k in range(16)]
                    o = _tail16(v)
                    for k in range(16):
                        buf[r, pl.ds(base + 16 * k, 16)] = o[k]
        else:
            init = tuple(jnp.zeros((16,), jnp.float32) for _ in range(_PPW))

            @plsc.parallel_loop(0, _T // 256, 1, unroll=1, carry=init)
            def _tail_acc(chunk, acc):
                base = chunk * 256
                outs = []
                for r in range(_LSTEP):
                    v = [buf[r, pl.ds(base + 16 * k, 16)] for k in range(16)]
                    outs.append(_tail16(v))
                new = []
                for i in range(_PPW):
                    local = None
                    for k in range(16):
                        dlt = outs[i][k] - outs[_PPW + i][k]
                        sq = dlt * dlt
                        local = sq if local is None else local + sq
                    new.append(acc[i] + local)
                return tuple(new)
            accs = _tail_acc

    lane = lax.iota(jnp.int32, 16)
    costv = jnp.zeros((16,), jnp.float32)
    inv_t = 1.0 / _T
    for i in range(_PPW):
        s_i = jnp.sum(accs[i]) * inv_t
        costv = costv + jnp.where(lane == i, s_i, 0.0)
    costref[...] = costv
    pltpu.sync_copy(costref, out_hbm.at[w])


def _sc_cost(px, py):
    mesh = plsc.VectorSubcoreMesh(core_axis_name="c", subcore_axis_name="s")
    fn = functools.partial(
        pl.kernel,
        mesh=mesh,
        out_type=jax.ShapeDtypeStruct((_NW, 16), jnp.float32),
        scratch_types=[
            pltpu.VMEM((_LSTEP, _T), jnp.float32),
            pltpu.VMEM((16,), jnp.float32),
            pltpu.SemaphoreType.DMA,
            pltpu.SemaphoreType.DMA,
        ],
        compiler_params=pltpu.CompilerParams(needs_layout_passes=False),
    )(_sc_cost_body)
    return fn(px, py)


def kernel(x, y, projections):
    costs = []
    for chunk in range(_NCHUNK):
        px, py = _project(x, y, projections, chunk)
        cost = _sc_cost(px, py)
        costs.append(cost[:, :_PPW].reshape(_BC, _NUM_PROJ))
    cost_bl = jnp.concatenate(costs, axis=0)
    return jnp.clip(jnp.mean(cost_bl, axis=-1), _EPS, None) ** 0.5
